# baseline (speedup 1.0000x reference)
import functools

import jax
import jax.numpy as jnp
from jax import lax
from jax.experimental import pallas as pl
from jax.experimental.pallas import tpu as pltpu
from jax.experimental.pallas import tpu_sc as plsc

N = 10000
E = 320000
D = 128
H = 128
C = 3

NC = 2
NS = 16
NW = NC * NS
L = 16
H2 = H // 2
CH = 128
CPT = 80
EPT = CH * CPT
EP = EPT * NW

NP = 10240
RPT = NP // NS
ZR = 128


def _lane_bcast(v, l):
    idx = jnp.full((L, 1), l, jnp.int32)
    dn = lax.GatherDimensionNumbers(
        offset_dims=(), collapsed_slice_dims=(0,), start_index_map=(0,))
    return lax.gather(v, idx, dn, (1,),
                      mode=lax.GatherScatterMode.PROMISE_IN_BOUNDS)


def _mesh():
    return plsc.VectorSubcoreMesh(core_axis_name="c", subcore_axis_name="s")


@functools.partial(
    pl.kernel,
    out_type=jax.ShapeDtypeStruct((NW, 1, N), jnp.float32),
    mesh=_mesh(),
    scratch_types=[
        pltpu.VMEM((CPT, 1, CH), jnp.int32),
        pltpu.VMEM((CPT, 1, CH), jnp.float32),
        pltpu.VMEM((N,), jnp.float32),
    ],
    compiler_params=pltpu.CompilerParams(needs_layout_passes=False),
)
def _sc_deg(col_hbm, w_hbm, deg_hbm, colv, wv, degv):
    c = lax.axis_index("c")
    s = lax.axis_index("s")
    wid = s * NC + c
    pltpu.sync_copy(col_hbm.at[wid], colv)
    pltpu.sync_copy(w_hbm.at[wid], wv)

    zeros = jnp.zeros((L,), jnp.float32)

    def zb(i, carry):
        degv[pl.ds(i * L, L)] = zeros
        return carry
    lax.fori_loop(0, N // L, zb, 0)

    def ebody(j, carry):
        def gbody(g, carry2):
            idx = colv[j, 0, pl.ds(g * L, L)]
            vals = wv[j, 0, pl.ds(g * L, L)]
            plsc.addupdate_scatter(degv, [idx], vals)
            return carry2
        lax.fori_loop(0, CH // L, gbody, 0)
        return carry
    lax.fori_loop(0, CPT, ebody, 0)
    pltpu.sync_copy(degv, deg_hbm.at[wid, 0])


@functools.partial(
    pl.kernel,
    out_type=jax.ShapeDtypeStruct((NC, 2, NP, H2), jnp.float32),
    mesh=_mesh(),
    scratch_types=[
        pltpu.VMEM((CPT, 1, CH), jnp.int32),
        pltpu.VMEM((CPT, 1, CH), jnp.int32),
        pltpu.VMEM((CPT, 1, CH), jnp.float32),
        pltpu.VMEM((CH, H2), jnp.float32),
        pltpu.VMEM((CH, H2), jnp.float32),
        pltpu.VMEM_SHARED((NP, H2), jnp.float32),
        pltpu.VMEM_SHARED((NP, H2), jnp.float32),
        pltpu.SemaphoreType.DMA,
        pltpu.SemaphoreType.DMA,
    ],
    compiler_params=pltpu.CompilerParams(
        needs_layout_passes=False, use_tc_tiling_on_sc=False),
)
def _sc_spmm(hs_hbm, row_hbm, col_hbm, w_hbm, z_hbm, out_hbm,
             rowv, colv, wv, msg0, msg1, hsS, accS, sem0, sem1):
    c = lax.axis_index("c")
    s = lax.axis_index("s")
    wid = s * NC + c

    pltpu.sync_copy(row_hbm.at[wid], rowv)
    pltpu.sync_copy(col_hbm.at[wid], colv)
    pltpu.sync_copy(w_hbm.at[wid], wv)

    for p in range(2):
        def zc(k, carry):
            sl = pl.ds(s * RPT + k * ZR, ZR)
            pltpu.sync_copy(hs_hbm.at[p].at[sl], hsS.at[sl])
            pltpu.sync_copy(z_hbm.at[sl], accS.at[sl])
            return carry
        lax.fori_loop(0, RPT // ZR, zc, 0)
        plsc.subcore_barrier()

        def scale_and_scatter(j, msg):
            def grp(g, carry2):
                w16 = wv[j, 0, pl.ds(g * L, L)]
                for l in range(L):
                    wj = _lane_bcast(w16, l)
                    jrow = g * L + l
                    for f in range(H2 // L):
                        sl = pl.ds(f * L, L)
                        msg[jrow, sl] = msg[jrow, sl] * wj
                return carry2
            lax.fori_loop(0, CH // L, grp, 0)
            pltpu.sync_copy(msg, accS.at[colv.at[j, 0]], add=True)

        pltpu.async_copy(hsS.at[rowv.at[0, 0]], msg0, sem0)

        def outer(j2, carry):
            j = 2 * j2
            pltpu.async_copy(hsS.at[rowv.at[j + 1, 0]], msg1, sem1)
            pltpu.make_async_copy(hsS.at[rowv.at[j, 0]], msg0, sem0).wait()
            scale_and_scatter(j, msg0)

            jn = lax.rem(j + 2, CPT)
            pltpu.async_copy(hsS.at[rowv.at[jn, 0]], msg0, sem0)
            pltpu.make_async_copy(hsS.at[rowv.at[j + 1, 0]], msg1, sem1).wait()
            scale_and_scatter(j + 1, msg1)
            return carry
        lax.fori_loop(0, CPT // 2, outer, 0)
        pltpu.make_async_copy(hsS.at[rowv.at[0, 0]], msg0, sem0).wait()
        plsc.subcore_barrier()

        def wb(k, carry):
            sl = pl.ds(s * RPT + k * ZR, ZR)
            pltpu.sync_copy(accS.at[sl], out_hbm.at[c, p].at[sl])
            return carry
        lax.fori_loop(0, RPT // ZR, wb, 0)


def _tc_prep_body(degp_ref, x_ref, w1_ref, dinv_ref, h_ref, hs_ref):
    deg = 1.0 + jnp.sum(degp_ref[...], axis=(0, 1))
    dinv = lax.rsqrt(deg)
    dinv_ref[...] = dinv[:, None]
    h = jnp.dot(x_ref[...], w1_ref[...], preferred_element_type=jnp.float32)
    h_ref[...] = h
    hs = h * dinv[:, None]
    hs_ref[0] = hs[:, :H2]
    hs_ref[1] = hs[:, H2:]


def _tc_mid_body(acc_ref, h_ref, dinv_ref, b_ref, w_ref, hn_ref, hsn_ref):
    dinv = dinv_ref[...]
    aggr = jnp.concatenate(
        [acc_ref[0, 0, :N] + acc_ref[1, 0, :N],
         acc_ref[0, 1, :N] + acc_ref[1, 1, :N]], axis=1)
    out = dinv * aggr + (dinv * dinv) * h_ref[...] + b_ref[...]
    out = jnp.maximum(out, 0.0)
    hn = jnp.dot(out, w_ref[...], preferred_element_type=jnp.float32)
    hn_ref[...] = hn
    hs = hn * dinv
    hsn_ref[0] = hs[:, :H2]
    hsn_ref[1] = hs[:, H2:]


def _tc_tail_body(acc_ref, h_ref, dinv_ref, b_ref, lw_ref, lb_ref, o_ref):
    dinv = dinv_ref[...]
    aggr = jnp.concatenate(
        [acc_ref[0, 0, :N] + acc_ref[1, 0, :N],
         acc_ref[0, 1, :N] + acc_ref[1, 1, :N]], axis=1)
    out = dinv * aggr + dinv * dinv * h_ref[...] + b_ref[...]
    g = jnp.mean(out, axis=0, keepdims=True)
    logits = jnp.dot(g, lw_ref[...], preferred_element_type=jnp.float32) + lb_ref[...]
    m = jnp.max(logits, axis=1, keepdims=True)
    e = jnp.exp(logits - m)
    o_ref[...] = e / jnp.sum(e, axis=1, keepdims=True)


def _tc_prep(deg_part, x, W1):
    return pl.pallas_call(
        _tc_prep_body,
        out_shape=[
            jax.ShapeDtypeStruct((N, 1), jnp.float32),
            jax.ShapeDtypeStruct((N, H), jnp.float32),
            jax.ShapeDtypeStruct((2, N, H2), jnp.float32),
        ],
    )(deg_part, x, W1)


def _tc_mid(acc, h, dinv2d, b, Wn):
    return pl.pallas_call(
        _tc_mid_body,
        out_shape=[
            jax.ShapeDtypeStruct((N, H), jnp.float32),
            jax.ShapeDtypeStruct((2, N, H2), jnp.float32),
        ],
    )(acc, h, dinv2d, b, Wn)


def _tc_tail(acc, h, dinv2d, b, lin_W, lin_b):
    return pl.pallas_call(
        _tc_tail_body,
        out_shape=jax.ShapeDtypeStruct((1, C), jnp.float32),
    )(acc, h, dinv2d, b, lin_W, lin_b)


def kernel(x, edge_index, edge_weight, W1, b1, W2, b2, W3, b3, lin_W, lin_b):
    row = edge_index[0]
    col = edge_index[1]
    pad = EP - E
    rowp = jnp.concatenate([row, jnp.zeros((pad,), row.dtype)]).reshape(NW, CPT, 1, CH)
    colp = jnp.concatenate([col, jnp.zeros((pad,), col.dtype)]).reshape(NW, CPT, 1, CH)
    wp = jnp.concatenate(
        [edge_weight, jnp.zeros((pad,), edge_weight.dtype)]).reshape(NW, CPT, 1, CH)

    zeros_np = jnp.zeros((NP, H2), jnp.float32)
    deg_part = _sc_deg(colp, wp)
    dinv2d, h1, hs1 = _tc_prep(deg_part, x, W1)
    acc1 = _sc_spmm(hs1, rowp, colp, wp, zeros_np)
    h2, hs2 = _tc_mid(acc1, h1, dinv2d, b1.reshape(1, H), W2)
    acc2 = _sc_spmm(hs2, rowp, colp, wp, zeros_np)
    h3, hs3 = _tc_mid(acc2, h2, dinv2d, b2.reshape(1, H), W3)
    acc3 = _sc_spmm(hs3, rowp, colp, wp, zeros_np)
    return _tc_tail(acc3, h3, dinv2d, b3.reshape(1, H), lin_W, lin_b.reshape(1, C))


# speedup vs baseline: 1.5896x; 1.5896x over previous
import functools

import jax
import jax.numpy as jnp
from jax import lax
from jax.experimental import pallas as pl
from jax.experimental.pallas import tpu as pltpu
from jax.experimental.pallas import tpu_sc as plsc

N = 10000
E = 320000
D = 128
H = 128
C = 3

NC = 2
NS = 16
NW = NC * NS
L = 16
H2 = H // 2
CH = 128
CPT = 80
EPT = CH * CPT
EP = EPT * NW

NP = 10240
RPT = NP // NS
ZR = 128


def _lane_bcast(v, l):
    idx = jnp.full((L, 1), l, jnp.int32)
    dn = lax.GatherDimensionNumbers(
        offset_dims=(), collapsed_slice_dims=(0,), start_index_map=(0,))
    return lax.gather(v, idx, dn, (1,),
                      mode=lax.GatherScatterMode.PROMISE_IN_BOUNDS)


def _mesh():
    return plsc.VectorSubcoreMesh(core_axis_name="c", subcore_axis_name="s")


@functools.partial(
    pl.kernel,
    out_type=jax.ShapeDtypeStruct((NW, 1, N), jnp.float32),
    mesh=_mesh(),
    scratch_types=[
        pltpu.VMEM((CPT, 1, CH), jnp.int32),
        pltpu.VMEM((CPT, 1, CH), jnp.float32),
        pltpu.VMEM((N,), jnp.float32),
    ],
    compiler_params=pltpu.CompilerParams(needs_layout_passes=False),
)
def _sc_deg(col_hbm, w_hbm, deg_hbm, colv, wv, degv):
    c = lax.axis_index("c")
    s = lax.axis_index("s")
    wid = s * NC + c
    pltpu.sync_copy(col_hbm.at[wid], colv)
    pltpu.sync_copy(w_hbm.at[wid], wv)

    zeros = jnp.zeros((L,), jnp.float32)

    def zb(i, carry):
        degv[pl.ds(i * L, L)] = zeros
        return carry
    lax.fori_loop(0, N // L, zb, 0)

    def ebody(j, carry):
        def gbody(g, carry2):
            idx = colv[j, 0, pl.ds(g * L, L)]
            vals = wv[j, 0, pl.ds(g * L, L)]
            plsc.addupdate_scatter(degv, [idx], vals)
            return carry2
        lax.fori_loop(0, CH // L, gbody, 0)
        return carry
    lax.fori_loop(0, CPT, ebody, 0)
    pltpu.sync_copy(degv, deg_hbm.at[wid, 0])


@functools.partial(
    pl.kernel,
    out_type=jax.ShapeDtypeStruct((NC, 2, NP, H2), jnp.float32),
    mesh=_mesh(),
    scratch_types=[
        pltpu.VMEM((CPT, 1, CH), jnp.int32),
        pltpu.VMEM((CPT, 1, CH), jnp.int32),
        pltpu.VMEM((CPT, 1, CH), jnp.float32),
        pltpu.VMEM((CH, H2), jnp.float32),
        pltpu.VMEM((CH, H2), jnp.float32),
        pltpu.VMEM_SHARED((NP, H2), jnp.float32),
        pltpu.VMEM_SHARED((NP, H2), jnp.float32),
        pltpu.SemaphoreType.DMA,
        pltpu.SemaphoreType.DMA,
    ],
    compiler_params=pltpu.CompilerParams(
        needs_layout_passes=False, use_tc_tiling_on_sc=False),
)
def _sc_spmm(hs_hbm, row_hbm, col_hbm, w_hbm, z_hbm, out_hbm,
             rowv, colv, wv, msg0, msg1, hsS, accS, sem0, sem1):
    c = lax.axis_index("c")
    s = lax.axis_index("s")
    wid = s * NC + c

    pltpu.sync_copy(row_hbm.at[wid], rowv)
    pltpu.sync_copy(col_hbm.at[wid], colv)
    pltpu.sync_copy(w_hbm.at[wid], wv)

    for p in range(2):
        def zc(k, carry):
            sl = pl.ds(s * RPT + k * ZR, ZR)
            pltpu.sync_copy(hs_hbm.at[p].at[sl], hsS.at[sl])
            pltpu.sync_copy(z_hbm.at[sl], accS.at[sl])
            return carry
        lax.fori_loop(0, RPT // ZR, zc, 0)
        plsc.subcore_barrier()

        def chunk(j, carry):
            pltpu.async_copy(hsS.at[rowv.at[j, 0]], msg0, sem0).wait()

            def grp(g, carry2):
                w16 = wv[j, 0, pl.ds(g * L, L)]
                for l in range(L):
                    wj = _lane_bcast(w16, l)
                    jrow = g * L + l
                    for f in range(H2 // L):
                        sl = pl.ds(f * L, L)
                        msg1[jrow, sl] = msg0[jrow, sl] * wj
                return carry2
            lax.fori_loop(0, CH // L, grp, 0)
            pltpu.sync_copy(msg1, accS.at[colv.at[j, 0]], add=True)
            return carry
        lax.fori_loop(0, CPT, chunk, 0)
        plsc.subcore_barrier()

        def wb(k, carry):
            sl = pl.ds(s * RPT + k * ZR, ZR)
            pltpu.sync_copy(accS.at[sl], out_hbm.at[c, p].at[sl])
            return carry
        lax.fori_loop(0, RPT // ZR, wb, 0)


def _tc_prep_body(degp_ref, x_ref, w1_ref, dinv_ref, h_ref, hs_ref):
    deg = 1.0 + jnp.sum(degp_ref[...], axis=(0, 1))
    dinv = lax.rsqrt(deg)
    dinv_ref[...] = dinv[:, None]
    h = jnp.dot(x_ref[...], w1_ref[...], preferred_element_type=jnp.float32)
    h_ref[...] = h
    hs = h * dinv[:, None]
    hs_ref[0] = hs[:, :H2]
    hs_ref[1] = hs[:, H2:]


def _tc_mid_body(acc_ref, h_ref, dinv_ref, b_ref, w_ref, hn_ref, hsn_ref):
    dinv = dinv_ref[...]
    aggr = jnp.concatenate(
        [acc_ref[0, 0, :N] + acc_ref[1, 0, :N],
         acc_ref[0, 1, :N] + acc_ref[1, 1, :N]], axis=1)
    out = dinv * aggr + (dinv * dinv) * h_ref[...] + b_ref[...]
    out = jnp.maximum(out, 0.0)
    hn = jnp.dot(out, w_ref[...], preferred_element_type=jnp.float32)
    hn_ref[...] = hn
    hs = hn * dinv
    hsn_ref[0] = hs[:, :H2]
    hsn_ref[1] = hs[:, H2:]


def _tc_tail_body(acc_ref, h_ref, dinv_ref, b_ref, lw_ref, lb_ref, o_ref):
    dinv = dinv_ref[...]
    aggr = jnp.concatenate(
        [acc_ref[0, 0, :N] + acc_ref[1, 0, :N],
         acc_ref[0, 1, :N] + acc_ref[1, 1, :N]], axis=1)
    out = dinv * aggr + dinv * dinv * h_ref[...] + b_ref[...]
    g = jnp.mean(out, axis=0, keepdims=True)
    logits = jnp.dot(g, lw_ref[...], preferred_element_type=jnp.float32) + lb_ref[...]
    m = jnp.max(logits, axis=1, keepdims=True)
    e = jnp.exp(logits - m)
    o_ref[...] = e / jnp.sum(e, axis=1, keepdims=True)


def _tc_prep(deg_part, x, W1):
    return pl.pallas_call(
        _tc_prep_body,
        out_shape=[
            jax.ShapeDtypeStruct((N, 1), jnp.float32),
            jax.ShapeDtypeStruct((N, H), jnp.float32),
            jax.ShapeDtypeStruct((2, N, H2), jnp.float32),
        ],
    )(deg_part, x, W1)


def _tc_mid(acc, h, dinv2d, b, Wn):
    return pl.pallas_call(
        _tc_mid_body,
        out_shape=[
            jax.ShapeDtypeStruct((N, H), jnp.float32),
            jax.ShapeDtypeStruct((2, N, H2), jnp.float32),
        ],
    )(acc, h, dinv2d, b, Wn)


def _tc_tail(acc, h, dinv2d, b, lin_W, lin_b):
    return pl.pallas_call(
        _tc_tail_body,
        out_shape=jax.ShapeDtypeStruct((1, C), jnp.float32),
    )(acc, h, dinv2d, b, lin_W, lin_b)


def kernel(x, edge_index, edge_weight, W1, b1, W2, b2, W3, b3, lin_W, lin_b):
    row = edge_index[0]
    col = edge_index[1]
    pad = EP - E
    rowp = jnp.concatenate([row, jnp.zeros((pad,), row.dtype)]).reshape(NW, CPT, 1, CH)
    colp = jnp.concatenate([col, jnp.zeros((pad,), col.dtype)]).reshape(NW, CPT, 1, CH)
    wp = jnp.concatenate(
        [edge_weight, jnp.zeros((pad,), edge_weight.dtype)]).reshape(NW, CPT, 1, CH)

    zeros_np = jnp.zeros((NP, H2), jnp.float32)
    deg_part = _sc_deg(colp, wp)
    dinv2d, h1, hs1 = _tc_prep(deg_part, x, W1)
    acc1 = _sc_spmm(hs1, rowp, colp, wp, zeros_np)
    h2, hs2 = _tc_mid(acc1, h1, dinv2d, b1.reshape(1, H), W2)
    acc2 = _sc_spmm(hs2, rowp, colp, wp, zeros_np)
    h3, hs3 = _tc_mid(acc2, h2, dinv2d, b2.reshape(1, H), W3)
    acc3 = _sc_spmm(hs3, rowp, colp, wp, zeros_np)
    return _tc_tail(acc3, h3, dinv2d, b3.reshape(1, H), lin_W, lin_b.reshape(1, C))


# speedup vs baseline: 1.6330x; 1.0273x over previous
import functools

import jax
import jax.numpy as jnp
from jax import lax
from jax.experimental import pallas as pl
from jax.experimental.pallas import tpu as pltpu
from jax.experimental.pallas import tpu_sc as plsc

N = 10000
E = 320000
D = 128
H = 128
C = 3

NC = 2
NS = 16
NW = NC * NS
L = 16
H2 = H // 2
CH = 128
CPT = 80
EPT = CH * CPT
EP = EPT * NW

NP = 10240
RPT = NP // NS
ZR = 128


def _lane_bcast(v, l):
    idx = jnp.full((L, 1), l, jnp.int32)
    dn = lax.GatherDimensionNumbers(
        offset_dims=(), collapsed_slice_dims=(0,), start_index_map=(0,))
    return lax.gather(v, idx, dn, (1,),
                      mode=lax.GatherScatterMode.PROMISE_IN_BOUNDS)


def _mesh():
    return plsc.VectorSubcoreMesh(core_axis_name="c", subcore_axis_name="s")


@functools.partial(
    pl.kernel,
    out_type=jax.ShapeDtypeStruct((NW, 1, N), jnp.float32),
    mesh=_mesh(),
    scratch_types=[
        pltpu.VMEM((CPT, 1, CH), jnp.int32),
        pltpu.VMEM((CPT, 1, CH), jnp.float32),
        pltpu.VMEM((N,), jnp.float32),
    ],
    compiler_params=pltpu.CompilerParams(needs_layout_passes=False),
)
def _sc_deg(col_hbm, w_hbm, deg_hbm, colv, wv, degv):
    c = lax.axis_index("c")
    s = lax.axis_index("s")
    wid = s * NC + c
    pltpu.sync_copy(col_hbm.at[wid], colv)
    pltpu.sync_copy(w_hbm.at[wid], wv)

    zeros = jnp.zeros((L,), jnp.float32)

    def zb(i, carry):
        degv[pl.ds(i * L, L)] = zeros
        return carry
    lax.fori_loop(0, N // L, zb, 0)

    def ebody(j, carry):
        def gbody(g, carry2):
            idx = colv[j, 0, pl.ds(g * L, L)]
            vals = wv[j, 0, pl.ds(g * L, L)]
            plsc.addupdate_scatter(degv, [idx], vals)
            return carry2
        lax.fori_loop(0, CH // L, gbody, 0)
        return carry
    lax.fori_loop(0, CPT, ebody, 0)
    pltpu.sync_copy(degv, deg_hbm.at[wid, 0])


@functools.partial(
    pl.kernel,
    out_type=jax.ShapeDtypeStruct((NC, 2, NP, H2), jnp.float32),
    mesh=_mesh(),
    scratch_types=[
        pltpu.VMEM((CPT, 1, CH), jnp.int32),
        pltpu.VMEM((CPT, 1, CH), jnp.int32),
        pltpu.VMEM((2, 1, CH), jnp.float32),
        pltpu.VMEM((CH, H2), jnp.float32),
        pltpu.VMEM((CH, H2), jnp.float32),
        pltpu.VMEM((CH, H2), jnp.float32),
        pltpu.VMEM_SHARED((NP, H2), jnp.float32),
        pltpu.VMEM_SHARED((NP, H2), jnp.float32),
        pltpu.SemaphoreType.DMA,
        pltpu.SemaphoreType.DMA,
        pltpu.SemaphoreType.DMA,
    ],
    compiler_params=pltpu.CompilerParams(
        needs_layout_passes=False, use_tc_tiling_on_sc=False),
)
def _sc_spmm(hs_hbm, row_hbm, col_hbm, w_hbm, z_hbm, out_hbm,
             rowv, colv, wring, msgA, msgB, msgC, hsS, accS,
             semG0, semG1, semW):
    c = lax.axis_index("c")
    s = lax.axis_index("s")
    wid = s * NC + c

    pltpu.sync_copy(row_hbm.at[wid], rowv)
    pltpu.sync_copy(col_hbm.at[wid], colv)

    for p in range(2):
        def zc(k, carry):
            sl = pl.ds(s * RPT + k * ZR, ZR)
            pltpu.sync_copy(hs_hbm.at[p].at[sl], hsS.at[sl])
            pltpu.sync_copy(z_hbm.at[sl], accS.at[sl])
            return carry
        lax.fori_loop(0, RPT // ZR, zc, 0)
        plsc.subcore_barrier()

        def scale(src, wslot):
            def grp(g, carry2):
                w16 = wring[wslot, 0, pl.ds(g * L, L)]
                for l in range(L):
                    wj = _lane_bcast(w16, l)
                    jrow = g * L + l
                    for f in range(H2 // L):
                        sl = pl.ds(f * L, L)
                        msgC[jrow, sl] = src[jrow, sl] * wj
                return carry2
            lax.fori_loop(0, CH // L, grp, 0)

        pltpu.async_copy(w_hbm.at[wid, 0], wring.at[0], semW)
        pltpu.async_copy(hsS.at[rowv.at[0, 0]], msgA, semG0)

        def outer(j2, carry):
            j = 2 * j2
            pltpu.async_copy(w_hbm.at[wid, j + 1], wring.at[1], semW)
            pltpu.async_copy(hsS.at[rowv.at[j + 1, 0]], msgB, semG1)
            pltpu.make_async_copy(hsS.at[rowv.at[j, 0]], msgA, semG0).wait()
            pltpu.make_async_copy(w_hbm.at[wid, j], wring.at[0], semW).wait()
            scale(msgA, 0)
            jn = lax.rem(j + 2, CPT)
            pltpu.async_copy(hsS.at[rowv.at[jn, 0]], msgA, semG0)
            pltpu.sync_copy(msgC, accS.at[colv.at[j, 0]], add=True)

            pltpu.async_copy(w_hbm.at[wid, jn], wring.at[0], semW)
            pltpu.make_async_copy(hsS.at[rowv.at[j + 1, 0]], msgB, semG1).wait()
            pltpu.make_async_copy(w_hbm.at[wid, j + 1], wring.at[1], semW).wait()
            scale(msgB, 1)
            pltpu.sync_copy(msgC, accS.at[colv.at[j + 1, 0]], add=True)
            return carry
        lax.fori_loop(0, CPT // 2, outer, 0)
        pltpu.make_async_copy(hsS.at[rowv.at[0, 0]], msgA, semG0).wait()
        pltpu.make_async_copy(w_hbm.at[wid, 0], wring.at[0], semW).wait()
        plsc.subcore_barrier()

        def wb(k, carry):
            sl = pl.ds(s * RPT + k * ZR, ZR)
            pltpu.sync_copy(accS.at[sl], out_hbm.at[c, p].at[sl])
            return carry
        lax.fori_loop(0, RPT // ZR, wb, 0)


def _tc_prep_body(degp_ref, x_ref, w1_ref, dinv_ref, h_ref, hs_ref):
    deg = 1.0 + jnp.sum(degp_ref[...], axis=(0, 1))
    dinv = lax.rsqrt(deg)
    dinv_ref[...] = dinv[:, None]
    h = jnp.dot(x_ref[...], w1_ref[...], preferred_element_type=jnp.float32)
    h_ref[...] = h
    hs = h * dinv[:, None]
    hs_ref[0] = hs[:, :H2]
    hs_ref[1] = hs[:, H2:]


def _tc_mid_body(acc_ref, h_ref, dinv_ref, b_ref, w_ref, hn_ref, hsn_ref):
    dinv = dinv_ref[...]
    aggr = jnp.concatenate(
        [acc_ref[0, 0, :N] + acc_ref[1, 0, :N],
         acc_ref[0, 1, :N] + acc_ref[1, 1, :N]], axis=1)
    out = dinv * aggr + (dinv * dinv) * h_ref[...] + b_ref[...]
    out = jnp.maximum(out, 0.0)
    hn = jnp.dot(out, w_ref[...], preferred_element_type=jnp.float32)
    hn_ref[...] = hn
    hs = hn * dinv
    hsn_ref[0] = hs[:, :H2]
    hsn_ref[1] = hs[:, H2:]


def _tc_tail_body(acc_ref, h_ref, dinv_ref, b_ref, lw_ref, lb_ref, o_ref):
    dinv = dinv_ref[...]
    aggr = jnp.concatenate(
        [acc_ref[0, 0, :N] + acc_ref[1, 0, :N],
         acc_ref[0, 1, :N] + acc_ref[1, 1, :N]], axis=1)
    out = dinv * aggr + dinv * dinv * h_ref[...] + b_ref[...]
    g = jnp.mean(out, axis=0, keepdims=True)
    logits = jnp.dot(g, lw_ref[...], preferred_element_type=jnp.float32) + lb_ref[...]
    m = jnp.max(logits, axis=1, keepdims=True)
    e = jnp.exp(logits - m)
    o_ref[...] = e / jnp.sum(e, axis=1, keepdims=True)


def _tc_prep(deg_part, x, W1):
    return pl.pallas_call(
        _tc_prep_body,
        out_shape=[
            jax.ShapeDtypeStruct((N, 1), jnp.float32),
            jax.ShapeDtypeStruct((N, H), jnp.float32),
            jax.ShapeDtypeStruct((2, N, H2), jnp.float32),
        ],
    )(deg_part, x, W1)


def _tc_mid(acc, h, dinv2d, b, Wn):
    return pl.pallas_call(
        _tc_mid_body,
        out_shape=[
            jax.ShapeDtypeStruct((N, H), jnp.float32),
            jax.ShapeDtypeStruct((2, N, H2), jnp.float32),
        ],
    )(acc, h, dinv2d, b, Wn)


def _tc_tail(acc, h, dinv2d, b, lin_W, lin_b):
    return pl.pallas_call(
        _tc_tail_body,
        out_shape=jax.ShapeDtypeStruct((1, C), jnp.float32),
    )(acc, h, dinv2d, b, lin_W, lin_b)


def kernel(x, edge_index, edge_weight, W1, b1, W2, b2, W3, b3, lin_W, lin_b):
    row = edge_index[0]
    col = edge_index[1]
    pad = EP - E
    rowp = jnp.concatenate([row, jnp.zeros((pad,), row.dtype)]).reshape(NW, CPT, 1, CH)
    colp = jnp.concatenate([col, jnp.zeros((pad,), col.dtype)]).reshape(NW, CPT, 1, CH)
    wp = jnp.concatenate(
        [edge_weight, jnp.zeros((pad,), edge_weight.dtype)]).reshape(NW, CPT, 1, CH)

    zeros_np = jnp.zeros((NP, H2), jnp.float32)
    deg_part = _sc_deg(colp, wp)
    dinv2d, h1, hs1 = _tc_prep(deg_part, x, W1)
    acc1 = _sc_spmm(hs1, rowp, colp, wp, zeros_np)
    h2, hs2 = _tc_mid(acc1, h1, dinv2d, b1.reshape(1, H), W2)
    acc2 = _sc_spmm(hs2, rowp, colp, wp, zeros_np)
    h3, hs3 = _tc_mid(acc2, h2, dinv2d, b2.reshape(1, H), W3)
    acc3 = _sc_spmm(hs3, rowp, colp, wp, zeros_np)
    return _tc_tail(acc3, h3, dinv2d, b3.reshape(1, H), lin_W, lin_b.reshape(1, C))
